# baseline (speedup 1.0000x reference)
import functools

import jax
import jax.numpy as jnp
from jax import lax
from jax.experimental import pallas as pl
from jax.experimental.pallas import tpu as pltpu
from jax.experimental.pallas import tpu_sc as plsc

K_IN, K_HID, K_OUT = 1.0, 1.1, 1.2

D = 128
NC, NS, LANES = 2, 16, 16
NW = NC * NS
B = 128
RPT = 640
RCH = RPT // B
HPAD = 10016
NPAD = NS * RPT


def _fill_rows(ref, n_rows, n_cols, value):
    v = jnp.full((16,), value, jnp.float32)

    def body(i, _):
        for c in range(n_cols // 16):
            ref[i, pl.ds(c * 16, 16)] = v
        return 0

    lax.fori_loop(0, n_rows, body, 0)


def _fill_iota_rows(idx_v, base):
    iot = lax.iota(jnp.int32, 16)
    for q in range(RCH):
        for k in range(B // 16):
            idx_v[q, pl.ds(k * 16, 16)] = iot + (base + q * B + k * 16)


def _deg_body(dst3, degp, dst_v, hist_v):
    c = lax.axis_index("c")
    s = lax.axis_index("s")
    wid = c * NS + s
    n_chunks = dst3.shape[1]
    pltpu.sync_copy(dst3.at[wid], dst_v)
    zv = jnp.zeros((16,), jnp.float32)

    def zb(i, _):
        hist_v[pl.ds(i * 16, 16)] = zv
        return 0

    lax.fori_loop(0, HPAD // 16, zb, 0)
    ones16 = jnp.full((16,), 1.0, jnp.float32)

    def chunk(j, _):
        for k in range(B // 16):
            idx = dst_v[j, pl.ds(k * 16, 16)]
            plsc.addupdate_scatter(hist_v, [idx], ones16)
        return 0

    lax.fori_loop(0, n_chunks, chunk, 0)
    pltpu.sync_copy(hist_v, degp.at[wid, 0])


def _make_agg_body(k0, k1):
    def _agg_body(ut, src3, dst3, part, src_v, dst_v, gbuf, idx_v, acc, sem):
        c = lax.axis_index("c")
        s = lax.axis_index("s")
        wid = c * NS + s
        pltpu.sync_copy(src3.at[wid], src_v)
        pltpu.sync_copy(dst3.at[wid], dst_v)
        _fill_rows(gbuf, B, D, 0.0)
        base = s * RPT
        _fill_iota_rows(idx_v, base)
        for q in range(RCH):
            pltpu.sync_copy(gbuf, acc.at[idx_v.at[q]])
        plsc.subcore_barrier()

        def chunk(j, _):
            pltpu.async_copy(ut.at[src_v.at[j]], gbuf, sem).wait()
            pltpu.sync_copy(gbuf, acc.at[dst_v.at[j]], add=True)
            return 0

        n_my = jnp.where(c == 0, k0, k1)
        lax.fori_loop(0, n_my, chunk, 0)
        plsc.subcore_barrier()
        for q in range(RCH):
            pltpu.async_copy(acc.at[idx_v.at[q]], gbuf, sem).wait()
            pltpu.sync_copy(gbuf, part.at[c, pl.ds(base + q * B, B)])

    return _agg_body


def _make_sc_kernels(n_chunks, k0, k1):
    mesh = plsc.VectorSubcoreMesh(core_axis_name="c", subcore_axis_name="s")
    deg_fn = pl.kernel(
        _deg_body,
        out_type=jax.ShapeDtypeStruct((NW, 1, HPAD), jnp.float32),
        mesh=mesh,
        compiler_params=pltpu.CompilerParams(needs_layout_passes=False),
        scratch_types=[
            pltpu.VMEM((n_chunks, B), jnp.int32),
            pltpu.VMEM((HPAD,), jnp.float32),
        ],
    )
    agg_fn = pl.kernel(
        _make_agg_body(k0, k1),
        out_type=jax.ShapeDtypeStruct((NC, NPAD, D), jnp.float32),
        mesh=mesh,
        scratch_types=[
            pltpu.VMEM((n_chunks, B), jnp.int32),
            pltpu.VMEM((n_chunks, B), jnp.int32),
            pltpu.VMEM((B, D), jnp.float32),
            pltpu.VMEM((RCH, B), jnp.int32),
            pltpu.VMEM_SHARED((NPAD, D), jnp.float32),
            pltpu.SemaphoreType.DMA,
        ],
    )
    return deg_fn, agg_fn


def _sinh(t):
    return 0.5 * (jnp.exp(t) - jnp.exp(-t))


def _acosh(t):
    return jnp.log(t + jnp.sqrt(t * t - 1.0))


def _expmap_proj(z, k):
    sqrtk = k ** 0.5
    n2 = jnp.sum(z * z, axis=1, keepdims=True)
    norm = jnp.clip(jnp.sqrt(n2 + 1e-12), 1e-7, None)
    xs = (sqrtk * _sinh(norm / sqrtk) / norm) * z
    x0 = jnp.sqrt(k + jnp.sum(xs * xs, axis=1, keepdims=True))
    col = lax.broadcasted_iota(jnp.int32, z.shape, 1)
    return jnp.where(col == 0, x0, xs)


def _logmap_tan(x, k):
    sqrtk = k ** 0.5
    col = lax.broadcasted_iota(jnp.int32, x.shape, 1)
    xs = jnp.where(col == 0, 0.0, x)
    norm = jnp.clip(jnp.sqrt(jnp.sum(xs * xs, axis=1, keepdims=True) + 1e-12), 1e-7, None)
    theta = _acosh(jnp.clip(x[:, 0:1] / sqrtk, 1.0 + 1e-7, None))
    return (sqrtk * theta / norm) * xs


def _dinv_from(dp):
    deg = jnp.sum(dp, axis=1, keepdims=True)
    return lax.rsqrt(deg)


def _pre1_body(x_ref, degp_ref, w_ref, b_ref, ut_ref):
    x = x_ref[...]
    sqrtk = K_IN ** 0.5
    n2 = jnp.sum(x * x, axis=1, keepdims=True)
    norm = jnp.clip(jnp.sqrt(n2 + 1e-12), 1e-7, None)
    xs = (sqrtk * _sinh(norm / sqrtk) / norm) * x
    x0 = jnp.sqrt(K_IN + jnp.sum(xs * xs, axis=1, keepdims=True))
    norm2 = jnp.clip(jnp.sqrt(jnp.sum(xs * xs, axis=1, keepdims=True) + 1e-12), 1e-7, None)
    theta = _acosh(jnp.clip(x0 / sqrtk, 1.0 + 1e-7, None))
    v1 = (sqrtk * theta / norm2) * xs
    hs = jnp.dot(v1, w_ref[...], preferred_element_type=jnp.float32) + b_ref[...]
    h = _expmap_proj(hs, K_IN)
    u1 = _logmap_tan(h, K_IN)
    ut_ref[...] = u1 * _dinv_from(degp_ref[...])


def _mid_body(p_ref, degp_ref, w_ref, b_ref, ut_ref):
    dinv = _dinv_from(degp_ref[...])
    pv = p_ref[...]
    agg = (pv[0] + pv[1]) * dinv
    z = jnp.maximum(agg, 0.0)
    o1 = _expmap_proj(z, K_HID)
    v2 = _logmap_tan(o1, K_HID)
    hs2 = jnp.dot(v2, w_ref[...], preferred_element_type=jnp.float32) + b_ref[...]
    h2 = _expmap_proj(hs2, K_HID)
    u2 = _logmap_tan(h2, K_HID)
    ut_ref[...] = u2 * dinv


def _post_body(p_ref, degp_ref, out_ref):
    dinv = _dinv_from(degp_ref[...])
    pv = p_ref[...]
    agg = (pv[0] + pv[1]) * dinv
    z = jnp.maximum(agg, 0.0)
    out_ref[...] = _expmap_proj(z, K_OUT)


def _tc_calls(n):
    rblk = 1000
    grid = (n // rblk,)
    row_spec = pl.BlockSpec((rblk, D), lambda i: (i, 0))
    degp_spec = pl.BlockSpec((rblk, NW), lambda i: (i, 0))
    part_spec = pl.BlockSpec((NC, rblk, D), lambda i: (0, i, 0))
    w_spec = pl.BlockSpec((D, D), lambda i: (0, 0))
    b_spec = pl.BlockSpec((1, D), lambda i: (0, 0))
    out_sd = jax.ShapeDtypeStruct((n, D), jnp.float32)
    pre1 = pl.pallas_call(
        _pre1_body, grid=grid,
        in_specs=[row_spec, degp_spec, w_spec, b_spec],
        out_specs=row_spec, out_shape=out_sd)
    mid = pl.pallas_call(
        _mid_body, grid=grid,
        in_specs=[part_spec, degp_spec, w_spec, b_spec],
        out_specs=row_spec, out_shape=out_sd)
    post = pl.pallas_call(
        _post_body, grid=grid,
        in_specs=[part_spec, degp_spec],
        out_specs=row_spec, out_shape=out_sd)
    return pre1, mid, post


def kernel(x, edge_index, W1, b1, W2, b2):
    n = x.shape[0]
    e = edge_index.shape[1]
    src = edge_index[0].astype(jnp.int32)
    dst = edge_index[1].astype(jnp.int32)
    nodes = jnp.arange(n, dtype=jnp.int32)
    e_tot = e + n
    ktot = -(-e_tot // (NS * B))
    k0 = (ktot * 58 + 50) // 100
    k1 = ktot - k0
    n_chunks = max(k0, k1)
    cap = NS * ktot * B
    srcp = jnp.concatenate(
        [src, nodes, jnp.zeros((cap - e_tot,), jnp.int32)])
    dstp = jnp.concatenate(
        [dst, nodes, jnp.full((cap - e_tot,), n, jnp.int32)])
    half = NS * k0 * B

    def _slab(flat, kc, fill):
        part = flat.reshape(NS, kc, B)
        if kc == n_chunks:
            return part
        pad = jnp.full((NS, n_chunks - kc, B), fill, jnp.int32)
        return jnp.concatenate([part, pad], axis=1)

    src3 = jnp.concatenate(
        [_slab(srcp[:half], k0, 0), _slab(srcp[half:], k1, 0)], axis=0)
    dst3 = jnp.concatenate(
        [_slab(dstp[:half], k0, n), _slab(dstp[half:], k1, n)], axis=0)

    W1p = jnp.pad(W1, ((0, 0), (1, 0)))
    b1p = jnp.pad(b1, (1, 0))[None, :]
    W2p = jnp.pad(W2, ((1, 0), (1, 0)))
    b2p = jnp.pad(b2, (1, 0))[None, :]

    deg_fn, agg_fn = _make_sc_kernels(n_chunks, k0, k1)
    pre1, mid, post = _tc_calls(n)

    degp = deg_fn(dst3)
    degt = jnp.transpose(degp[:, 0, :])
    ut1 = pre1(x, degt, W1p, b1p)
    p1 = agg_fn(ut1, src3, dst3)
    ut2 = mid(p1, degt, W2p, b2p)
    p2 = agg_fn(ut2, src3, dst3)
    return post(p2, degt)


# speedup vs baseline: 1.0119x; 1.0119x over previous
import functools

import jax
import jax.numpy as jnp
from jax import lax
from jax.experimental import pallas as pl
from jax.experimental.pallas import tpu as pltpu
from jax.experimental.pallas import tpu_sc as plsc

K_IN, K_HID, K_OUT = 1.0, 1.1, 1.2

D = 128
NC, NS, LANES = 2, 16, 16
NW = NC * NS
B = 128
RPT = 640
RCH = RPT // B
HPAD = 10016
NPAD = NS * RPT


def _fill_rows(ref, n_rows, n_cols, value):
    v = jnp.full((16,), value, jnp.float32)

    def body(i, _):
        for c in range(n_cols // 16):
            ref[i, pl.ds(c * 16, 16)] = v
        return 0

    lax.fori_loop(0, n_rows, body, 0)


def _fill_iota_rows(idx_v, base):
    iot = lax.iota(jnp.int32, 16)
    for q in range(RCH):
        for k in range(B // 16):
            idx_v[q, pl.ds(k * 16, 16)] = iot + (base + q * B + k * 16)


def _deg_body(dst3, degp, dst_v, hist_v):
    c = lax.axis_index("c")
    s = lax.axis_index("s")
    wid = c * NS + s
    n_chunks = dst3.shape[1]
    pltpu.sync_copy(dst3.at[wid], dst_v)
    zv = jnp.zeros((16,), jnp.float32)

    def zb(i, _):
        hist_v[pl.ds(i * 16, 16)] = zv
        return 0

    lax.fori_loop(0, HPAD // 16, zb, 0)
    ones16 = jnp.full((16,), 1.0, jnp.float32)

    def chunk(j, _):
        for k in range(B // 16):
            idx = dst_v[j, pl.ds(k * 16, 16)]
            plsc.addupdate_scatter(hist_v, [idx], ones16)
        return 0

    lax.fori_loop(0, n_chunks, chunk, 0)
    pltpu.sync_copy(hist_v, degp.at[wid, 0])


def _make_agg_body(k0, k1):
    def _agg_body(ut, src3, dst3, part, src_v, dst_v, gbuf, idx_v, acc, sem):
        c = lax.axis_index("c")
        s = lax.axis_index("s")
        wid = c * NS + s
        pltpu.sync_copy(src3.at[wid], src_v)
        pltpu.sync_copy(dst3.at[wid], dst_v)
        _fill_rows(gbuf, B, D, 0.0)
        base = s * RPT
        _fill_iota_rows(idx_v, base)
        for q in range(RCH):
            pltpu.sync_copy(gbuf, acc.at[idx_v.at[q]])
        plsc.subcore_barrier()

        def chunk(j, _):
            pltpu.async_copy(ut.at[src_v.at[j]], gbuf, sem).wait()
            pltpu.sync_copy(gbuf, acc.at[dst_v.at[j]], add=True)
            return 0

        n_my = jnp.where(c == 0, k0, k1)
        lax.fori_loop(0, n_my, chunk, 0)
        plsc.subcore_barrier()
        for q in range(RCH):
            pltpu.async_copy(acc.at[idx_v.at[q]], gbuf, sem).wait()
            pltpu.sync_copy(gbuf, part.at[c, pl.ds(base + q * B, B)])

    return _agg_body


def _make_sc_kernels(n_chunks, k0, k1):
    mesh = plsc.VectorSubcoreMesh(core_axis_name="c", subcore_axis_name="s")
    deg_fn = pl.kernel(
        _deg_body,
        out_type=jax.ShapeDtypeStruct((NW, 1, HPAD), jnp.float32),
        mesh=mesh,
        compiler_params=pltpu.CompilerParams(needs_layout_passes=False),
        scratch_types=[
            pltpu.VMEM((n_chunks, B), jnp.int32),
            pltpu.VMEM((HPAD,), jnp.float32),
        ],
    )
    agg_fn = pl.kernel(
        _make_agg_body(k0, k1),
        out_type=jax.ShapeDtypeStruct((NC, NPAD, D), jnp.float32),
        mesh=mesh,
        scratch_types=[
            pltpu.VMEM((n_chunks, B), jnp.int32),
            pltpu.VMEM((n_chunks, B), jnp.int32),
            pltpu.VMEM((B, D), jnp.float32),
            pltpu.VMEM((RCH, B), jnp.int32),
            pltpu.VMEM_SHARED((NPAD, D), jnp.float32),
            pltpu.SemaphoreType.DMA,
        ],
    )
    return deg_fn, agg_fn


def _sinh(t):
    return 0.5 * (jnp.exp(t) - jnp.exp(-t))


def _acosh(t):
    return jnp.log(t + jnp.sqrt(t * t - 1.0))


def _expmap_proj(z, k):
    sqrtk = k ** 0.5
    n2 = jnp.sum(z * z, axis=1, keepdims=True)
    norm = jnp.clip(jnp.sqrt(n2 + 1e-12), 1e-7, None)
    xs = (sqrtk * _sinh(norm / sqrtk) / norm) * z
    x0 = jnp.sqrt(k + jnp.sum(xs * xs, axis=1, keepdims=True))
    col = lax.broadcasted_iota(jnp.int32, z.shape, 1)
    return jnp.where(col == 0, x0, xs)


def _logmap_tan(x, k):
    sqrtk = k ** 0.5
    col = lax.broadcasted_iota(jnp.int32, x.shape, 1)
    xs = jnp.where(col == 0, 0.0, x)
    norm = jnp.clip(jnp.sqrt(jnp.sum(xs * xs, axis=1, keepdims=True) + 1e-12), 1e-7, None)
    theta = _acosh(jnp.clip(x[:, 0:1] / sqrtk, 1.0 + 1e-7, None))
    return (sqrtk * theta / norm) * xs


def _dinv_from(dp):
    deg = jnp.sum(dp, axis=1, keepdims=True)
    return lax.rsqrt(deg)


def _pre1_body(x_ref, degp_ref, w_ref, b_ref, ut_ref):
    x = x_ref[...]
    sqrtk = K_IN ** 0.5
    n2 = jnp.sum(x * x, axis=1, keepdims=True)
    norm = jnp.clip(jnp.sqrt(n2 + 1e-12), 1e-7, None)
    xs = (sqrtk * _sinh(norm / sqrtk) / norm) * x
    x0 = jnp.sqrt(K_IN + jnp.sum(xs * xs, axis=1, keepdims=True))
    norm2 = jnp.clip(jnp.sqrt(jnp.sum(xs * xs, axis=1, keepdims=True) + 1e-12), 1e-7, None)
    theta = _acosh(jnp.clip(x0 / sqrtk, 1.0 + 1e-7, None))
    v1 = (sqrtk * theta / norm2) * xs
    hs = jnp.dot(v1, w_ref[...], preferred_element_type=jnp.float32) + b_ref[...]
    h = _expmap_proj(hs, K_IN)
    u1 = _logmap_tan(h, K_IN)
    ut_ref[...] = u1 * _dinv_from(degp_ref[...])


def _mid_body(p_ref, degp_ref, w_ref, b_ref, ut_ref):
    dinv = _dinv_from(degp_ref[...])
    pv = p_ref[...]
    agg = (pv[0] + pv[1]) * dinv
    z = jnp.maximum(agg, 0.0)
    o1 = _expmap_proj(z, K_HID)
    v2 = _logmap_tan(o1, K_HID)
    hs2 = jnp.dot(v2, w_ref[...], preferred_element_type=jnp.float32) + b_ref[...]
    h2 = _expmap_proj(hs2, K_HID)
    u2 = _logmap_tan(h2, K_HID)
    ut_ref[...] = u2 * dinv


def _post_body(p_ref, degp_ref, out_ref):
    dinv = _dinv_from(degp_ref[...])
    pv = p_ref[...]
    agg = (pv[0] + pv[1]) * dinv
    z = jnp.maximum(agg, 0.0)
    out_ref[...] = _expmap_proj(z, K_OUT)


def _tc_calls(n):
    rblk = 1000
    grid = (n // rblk,)
    row_spec = pl.BlockSpec((rblk, D), lambda i: (i, 0))
    degp_spec = pl.BlockSpec((rblk, NW), lambda i: (i, 0))
    part_spec = pl.BlockSpec((NC, rblk, D), lambda i: (0, i, 0))
    w_spec = pl.BlockSpec((D, D), lambda i: (0, 0))
    b_spec = pl.BlockSpec((1, D), lambda i: (0, 0))
    out_sd = jax.ShapeDtypeStruct((n, D), jnp.float32)
    pre1 = pl.pallas_call(
        _pre1_body, grid=grid,
        in_specs=[row_spec, degp_spec, w_spec, b_spec],
        out_specs=row_spec, out_shape=out_sd)
    mid = pl.pallas_call(
        _mid_body, grid=grid,
        in_specs=[part_spec, degp_spec, w_spec, b_spec],
        out_specs=row_spec, out_shape=out_sd)
    post = pl.pallas_call(
        _post_body, grid=grid,
        in_specs=[part_spec, degp_spec],
        out_specs=row_spec, out_shape=out_sd)
    return pre1, mid, post


def kernel(x, edge_index, W1, b1, W2, b2):
    n = x.shape[0]
    e = edge_index.shape[1]
    src = edge_index[0].astype(jnp.int32)
    dst = edge_index[1].astype(jnp.int32)
    nodes = jnp.arange(n, dtype=jnp.int32)
    e_tot = e + n
    ktot = -(-e_tot // (NS * B))
    k0 = (ktot * 58) // 100
    k1 = ktot - k0
    n_chunks = max(k0, k1)
    cap = NS * ktot * B
    srcp = jnp.concatenate(
        [src, nodes, jnp.zeros((cap - e_tot,), jnp.int32)])
    dstp = jnp.concatenate(
        [dst, nodes, jnp.full((cap - e_tot,), n, jnp.int32)])
    half = NS * k0 * B

    def _slab(flat, kc, fill):
        part = flat.reshape(NS, kc, B)
        if kc == n_chunks:
            return part
        pad = jnp.full((NS, n_chunks - kc, B), fill, jnp.int32)
        return jnp.concatenate([part, pad], axis=1)

    src3 = jnp.concatenate(
        [_slab(srcp[:half], k0, 0), _slab(srcp[half:], k1, 0)], axis=0)
    dst3 = jnp.concatenate(
        [_slab(dstp[:half], k0, n), _slab(dstp[half:], k1, n)], axis=0)

    W1p = jnp.pad(W1, ((0, 0), (1, 0)))
    b1p = jnp.pad(b1, (1, 0))[None, :]
    W2p = jnp.pad(W2, ((1, 0), (1, 0)))
    b2p = jnp.pad(b2, (1, 0))[None, :]

    deg_fn, agg_fn = _make_sc_kernels(n_chunks, k0, k1)
    pre1, mid, post = _tc_calls(n)

    degp = deg_fn(dst3)
    degt = jnp.transpose(degp[:, 0, :])
    ut1 = pre1(x, degt, W1p, b1p)
    p1 = agg_fn(ut1, src3, dst3)
    ut2 = mid(p1, degt, W2p, b2p)
    p2 = agg_fn(ut2, src3, dst3)
    return post(p2, degt)
